# baseline (speedup 1.0000x reference)
import jax
import jax.numpy as jnp
from jax import lax
from jax.experimental import pallas as pl
from jax.experimental.pallas import tpu as pltpu
from jax.experimental.pallas import tpu_sc as plsc

D_MODEL = 768
N_PATCHES = 576
N_TOT = 577
BATCH = 128
NW = 32
LANES = 16
W = 384
WV = W // LANES
PB = N_PATCHES // 8
TB = BATCH // 8
TC = D_MODEL // W
UNITS = PB * TB * TC
UPW = UNITS // NW
NBUF = 2


def _sc_body(in_hbm, pos_hbm, row0_hbm, out_hbm,
             row0_v, row0_rep, in_bufs, pos_bufs, out_bufs,
             in_sems, pos_sems, out_sems):
    c = lax.axis_index("c")
    s = lax.axis_index("s")
    wid = s * 2 + c

    def unit(j):
        u = wid + NW * j
        pb = u // (TB * TC)
        rem = u - pb * (TB * TC)
        tb = rem // TC
        tc = rem - tb * TC
        return pb, tb, tc

    def in_src(j):
        pb, tb, tc = unit(j)
        return in_hbm.at[pl.ds(8 * tb, 8), pl.ds(8 * pb, 8), pl.ds(W * tc, W)]

    def pos_src(j):
        pb, _, tc = unit(j)
        return pos_hbm.at[pl.ds(8 * pb, 8), pl.ds(W * tc, W)]

    def out_dst(j):
        pb, tb, tc = unit(j)
        return out_hbm.at[pl.ds(8 * pb + 1, 8), pl.ds(8 * tb, 8), pl.ds(W * tc, W)]

    pltpu.sync_copy(row0_hbm, row0_v)
    for r in range(8):
        @plsc.parallel_loop(0, D_MODEL // LANES, unroll=4)
        def _rep(i):
            o = i * LANES
            row0_rep[r, pl.ds(o, LANES)] = row0_v[pl.ds(o, LANES)]

    @pl.when(wid < TB)
    def _cls_row():
        pltpu.sync_copy(row0_rep, out_hbm.at[0, pl.ds(8 * wid, 8), :])

    for ph in range(NBUF):
        pltpu.async_copy(in_src(ph), in_bufs.at[ph], in_sems.at[ph])
        pltpu.async_copy(pos_src(ph), pos_bufs.at[ph], pos_sems.at[ph])

    @pl.loop(0, UPW, step=NBUF)
    def _unit_loop(g):
        for ph in range(NBUF):
            j = g + ph
            in_buf = in_bufs.at[ph]
            pos_buf = pos_bufs.at[ph]
            out_buf = out_bufs.at[ph]
            pltpu.make_async_copy(in_src(j), in_buf, in_sems.at[ph]).wait()
            pltpu.make_async_copy(pos_src(j), pos_buf, pos_sems.at[ph]).wait()

            @pl.when(g > 0)
            def _wait_out():
                pltpu.make_async_copy(out_buf, out_dst(j), out_sems.at[ph]).wait()

            for pp in range(8):
                @plsc.parallel_loop(0, WV, unroll=2)
                def _add(i):
                    o = i * LANES
                    pv = pos_buf[pp, pl.ds(o, LANES)]
                    for bb in range(8):
                        out_buf[pp, bb, pl.ds(o, LANES)] = (
                            in_buf[bb, pp, pl.ds(o, LANES)] + pv)

            pltpu.async_copy(out_buf, out_dst(j), out_sems.at[ph])
            nj = jnp.minimum(j + NBUF, UPW - 1)
            pltpu.async_copy(in_src(nj), in_bufs.at[ph], in_sems.at[ph])
            pltpu.async_copy(pos_src(nj), pos_bufs.at[ph], pos_sems.at[ph])

    for ph in range(NBUF):
        pltpu.make_async_copy(in_src(UPW - 1), in_bufs.at[ph], in_sems.at[ph]).wait()
        pltpu.make_async_copy(pos_src(UPW - 1), pos_bufs.at[ph], pos_sems.at[ph]).wait()
        pltpu.make_async_copy(out_bufs.at[ph], out_dst(UPW - NBUF + ph), out_sems.at[ph]).wait()


def kernel(inputs, class_embed, pos_table):
    pos_sh = pos_table[1:]
    row0 = class_embed.reshape(D_MODEL) + pos_table[0]
    mesh = plsc.VectorSubcoreMesh(
        core_axis_name="c", subcore_axis_name="s", num_cores=2, num_subcores=16)
    out_phys = pl.kernel(
        _sc_body,
        out_type=jax.ShapeDtypeStruct((N_TOT, BATCH, D_MODEL), jnp.float32),
        mesh=mesh,
        scratch_types=[
            pltpu.VMEM((D_MODEL,), jnp.float32),
            pltpu.VMEM((8, D_MODEL), jnp.float32),
            pltpu.VMEM((NBUF, 8, 8, W), jnp.float32),
            pltpu.VMEM((NBUF, 8, W), jnp.float32),
            pltpu.VMEM((NBUF, 8, 8, W), jnp.float32),
            pltpu.SemaphoreType.DMA((NBUF,)),
            pltpu.SemaphoreType.DMA((NBUF,)),
            pltpu.SemaphoreType.DMA((NBUF,)),
        ],
    )(inputs, pos_sh, row0)
    return jnp.transpose(out_phys, (1, 0, 2))


# speedup vs baseline: 1.1507x; 1.1507x over previous
import jax
import jax.numpy as jnp
from jax import lax
from jax.experimental import pallas as pl
from jax.experimental.pallas import tpu as pltpu
from jax.experimental.pallas import tpu_sc as plsc

D_MODEL = 768
N_PATCHES = 576
N_TOT = 577
BATCH = 128
NW = 32
LANES = 16
W = 256
WV = W // LANES
PB = N_PATCHES // 8
TB = BATCH // 8
TC = D_MODEL // W
UNITS = PB * TB * TC
UPW = UNITS // NW
NIN = 4
NOUT = 2


def _sc_body(in_hbm, pos_hbm, row0_hbm, out_hbm,
             row0_v, row0_rep, in_bufs, pos_bufs, out_bufs,
             in_sems, pos_sems, out_sems):
    c = lax.axis_index("c")
    s = lax.axis_index("s")
    wid = s * 2 + c

    def unit(j):
        u = wid + NW * j
        pb = u // (TB * TC)
        rem = u - pb * (TB * TC)
        tb = rem // TC
        tc = rem - tb * TC
        return pb, tb, tc

    def in_src(j):
        pb, tb, tc = unit(j)
        return in_hbm.at[pl.ds(8 * tb, 8), pl.ds(8 * pb, 8), pl.ds(W * tc, W)]

    def pos_src(j):
        pb, _, tc = unit(j)
        return pos_hbm.at[pl.ds(8 * pb, 8), pl.ds(W * tc, W)]

    def out_dst(j):
        pb, tb, tc = unit(j)
        return out_hbm.at[pl.ds(8 * pb + 1, 8), pl.ds(8 * tb, 8), pl.ds(W * tc, W)]

    pltpu.sync_copy(row0_hbm, row0_v)
    for r in range(8):
        @plsc.parallel_loop(0, D_MODEL // LANES, unroll=4)
        def _rep(i):
            o = i * LANES
            row0_rep[r, pl.ds(o, LANES)] = row0_v[pl.ds(o, LANES)]

    @pl.when(wid < TB)
    def _cls_row():
        pltpu.sync_copy(row0_rep, out_hbm.at[0, pl.ds(8 * wid, 8), :])

    for ph in range(NIN):
        pltpu.async_copy(in_src(ph), in_bufs.at[ph], in_sems.at[ph])
        pltpu.async_copy(pos_src(ph), pos_bufs.at[ph], pos_sems.at[ph])

    @pl.loop(0, UPW, step=NIN)
    def _unit_loop(g):
        for ph in range(NIN):
            j = g + ph
            oph = ph % NOUT
            in_buf = in_bufs.at[ph]
            pos_buf = pos_bufs.at[ph]
            out_buf = out_bufs.at[oph]
            pltpu.make_async_copy(in_src(j), in_buf, in_sems.at[ph]).wait()
            pltpu.make_async_copy(pos_src(j), pos_buf, pos_sems.at[ph]).wait()

            @pl.when(j >= NOUT)
            def _wait_out():
                pltpu.make_async_copy(out_buf, out_dst(j), out_sems.at[oph]).wait()

            @plsc.parallel_loop(0, 8 * WV, unroll=2)
            def _add(i):
                pp = i >> 4
                o = (i & (WV - 1)) * LANES
                pv = pos_buf[pp, pl.ds(o, LANES)]
                for bb in range(8):
                    out_buf[pp, bb, pl.ds(o, LANES)] = (
                        in_buf[bb, pp, pl.ds(o, LANES)] + pv)

            pltpu.async_copy(out_buf, out_dst(j), out_sems.at[oph])
            nj = jnp.minimum(j + NIN, UPW - 1)
            pltpu.async_copy(in_src(nj), in_bufs.at[ph], in_sems.at[ph])
            pltpu.async_copy(pos_src(nj), pos_bufs.at[ph], pos_sems.at[ph])

    for ph in range(NIN):
        pltpu.make_async_copy(in_src(UPW - 1), in_bufs.at[ph], in_sems.at[ph]).wait()
        pltpu.make_async_copy(pos_src(UPW - 1), pos_bufs.at[ph], pos_sems.at[ph]).wait()
    for oph in range(NOUT):
        pltpu.make_async_copy(out_bufs.at[oph], out_dst(UPW - NOUT + oph), out_sems.at[oph]).wait()


def kernel(inputs, class_embed, pos_table):
    pos_sh = pos_table[1:]
    row0 = class_embed.reshape(D_MODEL) + pos_table[0]
    mesh = plsc.VectorSubcoreMesh(
        core_axis_name="c", subcore_axis_name="s", num_cores=2, num_subcores=16)
    out_phys = pl.kernel(
        _sc_body,
        out_type=jax.ShapeDtypeStruct((N_TOT, BATCH, D_MODEL), jnp.float32),
        mesh=mesh,
        scratch_types=[
            pltpu.VMEM((D_MODEL,), jnp.float32),
            pltpu.VMEM((8, D_MODEL), jnp.float32),
            pltpu.VMEM((NIN, 8, 8, W), jnp.float32),
            pltpu.VMEM((NIN, 8, W), jnp.float32),
            pltpu.VMEM((NOUT, 8, 8, W), jnp.float32),
            pltpu.SemaphoreType.DMA((NIN,)),
            pltpu.SemaphoreType.DMA((NIN,)),
            pltpu.SemaphoreType.DMA((NOUT,)),
        ],
    )(inputs, pos_sh, row0)
    return jnp.transpose(out_phys, (1, 0, 2))


# speedup vs baseline: 1.1872x; 1.0318x over previous
import jax
import jax.numpy as jnp
from jax import lax
from jax.experimental import pallas as pl
from jax.experimental.pallas import tpu as pltpu
from jax.experimental.pallas import tpu_sc as plsc

D_MODEL = 768
N_PATCHES = 576
N_TOT = 577
BATCH = 128
NW = 32
LANES = 16
W = 256
WV = W // LANES
PB = N_PATCHES // 8
TB = BATCH // 8
TC = D_MODEL // W
UNITS = PB * TB * TC
UPW = UNITS // NW
NIN = 4
NOUT = 2


def _sc_body(in_hbm, pos_hbm, row0_hbm, out_hbm,
             row0_v, row0_rep, in_bufs, pos_bufs, out_bufs, pos_sm,
             in_sems, pos_sems, out_sems):
    c = lax.axis_index("c")
    s = lax.axis_index("s")
    wid = s * 2 + c

    HPB = PB // 2
    for k in range(3):
        kb = s + 16 * k
        @pl.when(kb < HPB)
        def _fill():
            pltpu.sync_copy(pos_hbm.at[pl.ds(8 * (c * HPB + kb), 8), :],
                            pos_sm.at[pl.ds(8 * kb, 8), :])
    plsc.subcore_barrier()

    def unit(j):
        u = s + 16 * j
        pbl = u // (TB * TC)
        rem = u - pbl * (TB * TC)
        tb = rem // TC
        tc = rem - tb * TC
        return c * HPB + pbl, pbl, tb, tc

    def in_src(j):
        pb, _, tb, tc = unit(j)
        return in_hbm.at[pl.ds(8 * tb, 8), pl.ds(8 * pb, 8), pl.ds(W * tc, W)]

    def pos_src(j):
        _, pbl, _, tc = unit(j)
        return pos_sm.at[pl.ds(8 * pbl, 8), pl.ds(W * tc, W)]

    def out_dst(j):
        pb, _, tb, tc = unit(j)
        return out_hbm.at[pl.ds(8 * pb + 1, 8), pl.ds(8 * tb, 8), pl.ds(W * tc, W)]

    pltpu.sync_copy(row0_hbm, row0_v)
    for r in range(8):
        @plsc.parallel_loop(0, D_MODEL // LANES, unroll=4)
        def _rep(i):
            o = i * LANES
            row0_rep[r, pl.ds(o, LANES)] = row0_v[pl.ds(o, LANES)]

    @pl.when(wid < TB)
    def _cls_row():
        pltpu.sync_copy(row0_rep, out_hbm.at[0, pl.ds(8 * wid, 8), :])

    for ph in range(NIN):
        pltpu.async_copy(in_src(ph), in_bufs.at[ph], in_sems.at[ph])
        pltpu.async_copy(pos_src(ph), pos_bufs.at[ph], pos_sems.at[ph])

    @pl.loop(0, UPW, step=NIN)
    def _unit_loop(g):
        for ph in range(NIN):
            j = g + ph
            oph = ph % NOUT
            in_buf = in_bufs.at[ph]
            pos_buf = pos_bufs.at[ph]
            out_buf = out_bufs.at[oph]
            pltpu.make_async_copy(in_src(j), in_buf, in_sems.at[ph]).wait()
            pltpu.make_async_copy(pos_src(j), pos_buf, pos_sems.at[ph]).wait()

            @pl.when(j >= NOUT)
            def _wait_out():
                pltpu.make_async_copy(out_buf, out_dst(j), out_sems.at[oph]).wait()

            @plsc.parallel_loop(0, 8 * WV, unroll=2)
            def _add(i):
                pp = i >> 4
                o = (i & (WV - 1)) * LANES
                pv = pos_buf[pp, pl.ds(o, LANES)]
                for bb in range(8):
                    out_buf[pp, bb, pl.ds(o, LANES)] = (
                        in_buf[bb, pp, pl.ds(o, LANES)] + pv)

            pltpu.async_copy(out_buf, out_dst(j), out_sems.at[oph])
            nj = jnp.minimum(j + NIN, UPW - 1)
            pltpu.async_copy(in_src(nj), in_bufs.at[ph], in_sems.at[ph])
            pltpu.async_copy(pos_src(nj), pos_bufs.at[ph], pos_sems.at[ph])

    for ph in range(NIN):
        pltpu.make_async_copy(in_src(UPW - 1), in_bufs.at[ph], in_sems.at[ph]).wait()
        pltpu.make_async_copy(pos_src(UPW - 1), pos_bufs.at[ph], pos_sems.at[ph]).wait()
    for oph in range(NOUT):
        pltpu.make_async_copy(out_bufs.at[oph], out_dst(UPW - NOUT + oph), out_sems.at[oph]).wait()


def kernel(inputs, class_embed, pos_table):
    pos_sh = pos_table[1:]
    row0 = class_embed.reshape(D_MODEL) + pos_table[0]
    mesh = plsc.VectorSubcoreMesh(
        core_axis_name="c", subcore_axis_name="s", num_cores=2, num_subcores=16)
    out_phys = pl.kernel(
        _sc_body,
        out_type=jax.ShapeDtypeStruct((N_TOT, BATCH, D_MODEL), jnp.float32),
        mesh=mesh,
        scratch_types=[
            pltpu.VMEM((D_MODEL,), jnp.float32),
            pltpu.VMEM((8, D_MODEL), jnp.float32),
            pltpu.VMEM((NIN, 8, 8, W), jnp.float32),
            pltpu.VMEM((NIN, 8, W), jnp.float32),
            pltpu.VMEM((NOUT, 8, 8, W), jnp.float32),
            pltpu.VMEM_SHARED((N_PATCHES // 2, D_MODEL), jnp.float32),
            pltpu.SemaphoreType.DMA((NIN,)),
            pltpu.SemaphoreType.DMA((NIN,)),
            pltpu.SemaphoreType.DMA((NOUT,)),
        ],
    )(inputs, pos_sh, row0)
    return jnp.transpose(out_phys, (1, 0, 2))
